# Initial kernel scaffold; baseline (speedup 1.0000x reference)
import jax
import jax.numpy as jnp
from jax.experimental import pallas as pl


def kernel(image_batch):
    raise NotImplementedError("write your pallas kernel here")


# speedup vs baseline: 8.0720x; 8.0720x over previous
import math

import jax
import jax.numpy as jnp
import numpy as np
from jax.experimental import pallas as pl
from jax.experimental.pallas import tpu as pltpu

_PS = 65
_NB_ANG = 8
_NB_SP = 4
_CLIP = 0.2
_EPS = 1e-10
_KS = 26
_STRIDE = 16
_PAD = 6


def _np_consts(n_w):
    x = np.arange(_PS, dtype=np.float64) - _PS // 2
    sigma = _PS / math.sqrt(2.0)
    g = np.exp(-(x * x) / (2.0 * sigma * sigma))
    g = g / g.sum()
    gk = np.outer(g, g)
    gk_big = np.tile(gk, (1, n_w)).astype(np.float32)

    ks2 = _KS / 2.0
    xc2 = ks2 - np.abs(np.arange(_KS, dtype=np.float64) + 0.5 - ks2)
    w1 = xc2 / ks2
    wmat = np.zeros((_NB_SP, _PS))
    for s in range(_NB_SP):
        start = s * _STRIDE - _PAD
        for k in range(_KS):
            j = start + k
            if 0 <= j < _PS:
                wmat[s, j] = w1[k]
    wc_big = np.zeros((n_w * _PS, n_w * _NB_SP))
    for p in range(n_w):
        wc_big[p * _PS:(p + 1) * _PS, p * _NB_SP:(p + 1) * _NB_SP] = wmat.T
    wr = wmat.astype(np.float32)

    gsum = np.kron(np.eye(n_w), np.ones((_NB_SP, _NB_SP))).astype(np.float32)
    return gk_big, wc_big.astype(np.float32), wr, gsum


def _sift_body(x_ref, gk_ref, wc_ref, wr_ref, gs_ref, o_ref):
    x = x_ref[0, :, 0]
    c_dim = x.shape[0]
    w = x.shape[2]

    col = jax.lax.broadcasted_iota(jnp.int32, (_PS, w), 1)
    jj = jax.lax.rem(col, _PS)
    xr = jnp.concatenate([x[:, :, 1:], x[:, :, -1:]], axis=2)
    xl = jnp.concatenate([x[:, :, :1], x[:, :, :-1]], axis=2)
    right_val = jnp.where((jj == _PS - 1)[None], x, xr)
    left_val = jnp.where((jj == 0)[None], x, xl)
    gx = (right_val - left_val) * 0.5
    xd = jnp.concatenate([x[:, 1:, :], x[:, -1:, :]], axis=1)
    xu = jnp.concatenate([x[:, :1, :], x[:, :-1, :]], axis=1)
    gy = (xd - xu) * 0.5

    mag = jnp.sqrt(gx * gx + gy * gy + _EPS) * gk_ref[...][None]
    ori = jnp.arctan2(gy, gx + _EPS) + 2.0 * math.pi

    o_big = (_NB_ANG / (2.0 * math.pi)) * ori
    bo0f = jnp.floor(o_big)
    wo1 = o_big - bo0f
    b0 = jnp.where(bo0f >= _NB_ANG, bo0f - _NB_ANG, bo0f)
    c1 = wo1 * mag
    c0 = mag - c1

    wc = wc_ref[...]
    wr = wr_ref[...]
    pooled = []
    for a in range(_NB_ANG):
        am1 = (a - 1) % _NB_ANG
        contrib = (jnp.where(b0 == float(a), c0, 0.0)
                   + jnp.where(b0 == float(am1), c1, 0.0))
        for c in range(c_dim):
            t = jnp.dot(contrib[c], wc, preferred_element_type=jnp.float32)
            r = jnp.dot(wr, t, preferred_element_type=jnp.float32)
            pooled.append(r)
    p = jnp.stack(pooled).reshape(_NB_ANG, c_dim, _NB_SP, wc.shape[1])

    def _norm_sq(v):
        sq = v * v
        tot = jnp.sum(sq, axis=(0, 2))
        tot = jnp.dot(tot, gs_ref[...], preferred_element_type=jnp.float32)
        return tot[None, :, None, :]

    d = p / jnp.maximum(jnp.sqrt(_norm_sq(p)), 1e-12)
    d = jnp.clip(d, 0.0, _CLIP)
    d = d / jnp.maximum(jnp.sqrt(_norm_sq(d)), 1e-12)
    o_ref[...] = d[None, None]


def kernel(image_batch):
    b, c, h, w = image_batch.shape
    n_h, n_w = h // _PS, w // _PS
    m = _NB_SP * n_w

    gk_big, wc_big, wr, gsum = _np_consts(n_w)
    gk_big = jnp.asarray(gk_big)
    wc_big = jnp.asarray(wc_big)
    wr = jnp.asarray(wr)
    gsum = jnp.asarray(gsum)

    image5 = image_batch.reshape(b, c, n_h, _PS, w)
    out = pl.pallas_call(
        _sift_body,
        grid=(b, n_h),
        in_specs=[
            pl.BlockSpec((1, c, 1, _PS, w), lambda i, j: (i, 0, j, 0, 0)),
            pl.BlockSpec((_PS, w), lambda i, j: (0, 0)),
            pl.BlockSpec((w, m), lambda i, j: (0, 0)),
            pl.BlockSpec((_NB_SP, _PS), lambda i, j: (0, 0)),
            pl.BlockSpec((m, m), lambda i, j: (0, 0)),
        ],
        out_specs=pl.BlockSpec((1, 1, _NB_ANG, c, _NB_SP, m),
                               lambda i, j: (i, j, 0, 0, 0, 0)),
        out_shape=jax.ShapeDtypeStruct((b, n_h, _NB_ANG, c, _NB_SP, m),
                                       jnp.float32),
        compiler_params=pltpu.CompilerParams(
            dimension_semantics=("parallel", "arbitrary"),
        ),
        name="sift_descriptor",
    )(image5, gk_big, wc_big, wr, gsum)

    out = out.reshape(b, n_h, _NB_ANG, c, _NB_SP, n_w, _NB_SP)
    out = out.transpose(0, 3, 1, 5, 2, 4, 6)
    return out.reshape(b * n_h * n_w, c, _NB_ANG * _NB_SP * _NB_SP)


# speedup vs baseline: 8.3294x; 1.0319x over previous
import math

import jax
import jax.numpy as jnp
import numpy as np
from jax.experimental import pallas as pl
from jax.experimental.pallas import tpu as pltpu

_PS = 65
_NB_ANG = 8
_NB_SP = 4
_CLIP = 0.2
_EPS = 1e-10
_KS = 26
_STRIDE = 16
_PAD = 6


def _np_consts(n_w):
    x = np.arange(_PS, dtype=np.float64) - _PS // 2
    sigma = _PS / math.sqrt(2.0)
    g = np.exp(-(x * x) / (2.0 * sigma * sigma))
    g = g / g.sum()
    gk = np.outer(g, g)
    gk_big = np.tile(gk, (1, n_w)).astype(np.float32)

    ks2 = _KS / 2.0
    xc2 = ks2 - np.abs(np.arange(_KS, dtype=np.float64) + 0.5 - ks2)
    w1 = xc2 / ks2
    wmat = np.zeros((_NB_SP, _PS))
    for s in range(_NB_SP):
        start = s * _STRIDE - _PAD
        for k in range(_KS):
            j = start + k
            if 0 <= j < _PS:
                wmat[s, j] = w1[k]
    wc_big = np.zeros((n_w * _PS, n_w * _NB_SP))
    for p in range(n_w):
        wc_big[p * _PS:(p + 1) * _PS, p * _NB_SP:(p + 1) * _NB_SP] = wmat.T
    wr = wmat.astype(np.float32)

    gsum = np.kron(np.eye(n_w), np.ones((_NB_SP, _NB_SP))).astype(np.float32)
    return gk_big, wc_big.astype(np.float32), wr, gsum


def _sift_body(x_ref, gk_ref, wc_ref, wr_ref, gs_ref, o_ref):
    x = x_ref[0, :, 0]
    c_dim = x.shape[0]
    w = x.shape[2]

    col = jax.lax.broadcasted_iota(jnp.int32, (_PS, w), 1)
    jj = jax.lax.rem(col, _PS)
    xr = jnp.concatenate([x[:, :, 1:], x[:, :, -1:]], axis=2)
    xl = jnp.concatenate([x[:, :, :1], x[:, :, :-1]], axis=2)
    right_val = jnp.where((jj == _PS - 1)[None], x, xr)
    left_val = jnp.where((jj == 0)[None], x, xl)
    gx = (right_val - left_val) * 0.5
    xd = jnp.concatenate([x[:, 1:, :], x[:, -1:, :]], axis=1)
    xu = jnp.concatenate([x[:, :1, :], x[:, :-1, :]], axis=1)
    gy = (xd - xu) * 0.5

    mag = jnp.sqrt(gx * gx + gy * gy + _EPS) * gk_ref[...][None]

    gxe = gx + _EPS
    ax = jnp.abs(gxe)
    ay = jnp.abs(gy)
    mn = jnp.minimum(ax, ay)
    mx = jnp.maximum(ax, ay)
    r = mn / jnp.maximum(mx, 1e-30)
    r2 = r * r
    t = r * (1.2732134400
             + r2 * (-0.4235559358
                     + r2 * (0.2466700370
                             + r2 * (-0.1487908832
                                     + r2 * (0.0675856331
                                             + r2 * -0.0151248217)))))
    a1 = jnp.where(ay > ax, 2.0 - t, t)
    a2 = jnp.where(gxe < 0.0, 4.0 - a1, a1)
    u8 = jnp.where(gy < 0.0, 8.0 - a2, a2)
    bo0f = jnp.floor(u8)
    wo1 = u8 - bo0f
    b0 = jnp.where(bo0f >= _NB_ANG, bo0f - _NB_ANG, bo0f)
    c1 = wo1 * mag
    c0 = mag - c1

    wc = wc_ref[...]
    wr = wr_ref[...]
    pooled = []
    for a in range(_NB_ANG):
        am1 = (a - 1) % _NB_ANG
        contrib = (jnp.where(b0 == float(a), c0, 0.0)
                   + jnp.where(b0 == float(am1), c1, 0.0))
        contrib = contrib.astype(jnp.bfloat16)
        for c in range(c_dim):
            tc = jnp.dot(contrib[c], wc, preferred_element_type=jnp.float32)
            rc = jnp.dot(wr, tc, preferred_element_type=jnp.float32)
            pooled.append(rc)
    p = jnp.stack(pooled).reshape(_NB_ANG, c_dim, _NB_SP, wc.shape[1])

    def _norm_sq(v):
        sq = v * v
        tot = jnp.sum(sq, axis=(0, 2))
        tot = jnp.dot(tot, gs_ref[...], preferred_element_type=jnp.float32)
        return tot[None, :, None, :]

    d = p / jnp.maximum(jnp.sqrt(_norm_sq(p)), 1e-12)
    d = jnp.clip(d, 0.0, _CLIP)
    d = d / jnp.maximum(jnp.sqrt(_norm_sq(d)), 1e-12)
    o_ref[...] = d[None, None]


def kernel(image_batch):
    b, c, h, w = image_batch.shape
    n_h, n_w = h // _PS, w // _PS
    m = _NB_SP * n_w

    gk_big, wc_big, wr, gsum = _np_consts(n_w)
    gk_big = jnp.asarray(gk_big)
    wc_big = jnp.asarray(wc_big).astype(jnp.bfloat16)
    wr = jnp.asarray(wr)
    gsum = jnp.asarray(gsum)

    image5 = image_batch.reshape(b, c, n_h, _PS, w)
    out = pl.pallas_call(
        _sift_body,
        grid=(b, n_h),
        in_specs=[
            pl.BlockSpec((1, c, 1, _PS, w), lambda i, j: (i, 0, j, 0, 0)),
            pl.BlockSpec((_PS, w), lambda i, j: (0, 0)),
            pl.BlockSpec((w, m), lambda i, j: (0, 0)),
            pl.BlockSpec((_NB_SP, _PS), lambda i, j: (0, 0)),
            pl.BlockSpec((m, m), lambda i, j: (0, 0)),
        ],
        out_specs=pl.BlockSpec((1, 1, _NB_ANG, c, _NB_SP, m),
                               lambda i, j: (i, j, 0, 0, 0, 0)),
        out_shape=jax.ShapeDtypeStruct((b, n_h, _NB_ANG, c, _NB_SP, m),
                                       jnp.float32),
        compiler_params=pltpu.CompilerParams(
            dimension_semantics=("parallel", "arbitrary"),
        ),
        name="sift_descriptor",
    )(image5, gk_big, wc_big, wr, gsum)

    out = out.reshape(b, n_h, _NB_ANG, c, _NB_SP, n_w, _NB_SP)
    out = out.transpose(0, 3, 1, 5, 2, 4, 6)
    return out.reshape(b * n_h * n_w, c, _NB_ANG * _NB_SP * _NB_SP)


# speedup vs baseline: 9.3833x; 1.1265x over previous
import math

import jax
import jax.numpy as jnp
import numpy as np
from jax.experimental import pallas as pl
from jax.experimental.pallas import tpu as pltpu

_PS = 65
_NB_ANG = 8
_NB_SP = 4
_CLIP = 0.2
_EPS = 1e-10
_KS = 26
_STRIDE = 16
_PAD = 6


def _np_consts(n_w):
    x = np.arange(_PS, dtype=np.float64) - _PS // 2
    sigma = _PS / math.sqrt(2.0)
    g = np.exp(-(x * x) / (2.0 * sigma * sigma))
    g = g / g.sum()
    gk = np.outer(g, g)
    gk_big = np.tile(gk, (1, n_w)).astype(np.float32)

    ks2 = _KS / 2.0
    xc2 = ks2 - np.abs(np.arange(_KS, dtype=np.float64) + 0.5 - ks2)
    w1 = xc2 / ks2
    wmat = np.zeros((_NB_SP, _PS))
    for s in range(_NB_SP):
        start = s * _STRIDE - _PAD
        for k in range(_KS):
            j = start + k
            if 0 <= j < _PS:
                wmat[s, j] = w1[k]
    wc_big = np.zeros((n_w * _PS, n_w * _NB_SP))
    for p in range(n_w):
        wc_big[p * _PS:(p + 1) * _PS, p * _NB_SP:(p + 1) * _NB_SP] = wmat.T
    wr = wmat.astype(np.float32)

    gsum = np.kron(np.eye(n_w), np.ones((_NB_SP, _NB_SP))).astype(np.float32)
    return gk_big, wc_big.astype(np.float32), wr, gsum


def _sift_body(x_ref, gk_ref, wc_ref, wr_ref, o_ref):
    x = x_ref[0, :, 0]
    c_dim = x.shape[0]
    w = x.shape[2]

    col = jax.lax.broadcasted_iota(jnp.int32, (_PS, w), 1)
    jj = jax.lax.rem(col, _PS)
    xr = jnp.concatenate([x[:, :, 1:], x[:, :, -1:]], axis=2)
    xl = jnp.concatenate([x[:, :, :1], x[:, :, :-1]], axis=2)
    right_val = jnp.where((jj == _PS - 1)[None], x, xr)
    left_val = jnp.where((jj == 0)[None], x, xl)
    gx = (right_val - left_val) * 0.5
    xd = jnp.concatenate([x[:, 1:, :], x[:, -1:, :]], axis=1)
    xu = jnp.concatenate([x[:, :1, :], x[:, :-1, :]], axis=1)
    gy = (xd - xu) * 0.5

    mag = jnp.sqrt(gx * gx + gy * gy + _EPS) * gk_ref[...][None]

    gxe = gx + _EPS
    ax = jnp.abs(gxe)
    ay = jnp.abs(gy)
    mn = jnp.minimum(ax, ay)
    mx = jnp.maximum(ax, ay)
    r = mn / jnp.maximum(mx, 1e-30)
    r2 = r * r
    t = r * (1.2732134400
             + r2 * (-0.4235559358
                     + r2 * (0.2466700370
                             + r2 * (-0.1487908832
                                     + r2 * (0.0675856331
                                             + r2 * -0.0151248217)))))
    a1 = jnp.where(ay > ax, 2.0 - t, t)
    a2 = jnp.where(gxe < 0.0, 4.0 - a1, a1)
    u8 = jnp.where(gy < 0.0, 8.0 - a2, a2)
    bo0f = jnp.floor(u8)
    wo1 = u8 - bo0f
    b0 = jnp.where(bo0f >= _NB_ANG, bo0f - _NB_ANG, bo0f)
    c1 = wo1 * mag
    c0 = mag - c1

    wc = wc_ref[...]
    wr = wr_ref[...]
    pooled = {}
    for a in range(_NB_ANG):
        am1 = (a - 1) % _NB_ANG
        contrib = (jnp.where(b0 == float(a), c0, 0.0)
                   + jnp.where(b0 == float(am1), c1, 0.0))
        contrib = contrib.astype(jnp.bfloat16)
        for c in range(c_dim):
            tc = jnp.dot(contrib[c], wc, preferred_element_type=jnp.float32)
            rc = jnp.dot(wr, tc, preferred_element_type=jnp.float32)
            pooled[(a, c)] = rc
    q = jnp.stack([jnp.concatenate([pooled[(a, c)] for a in range(_NB_ANG)],
                                   axis=0) for c in range(c_dim)])

    qt = jnp.transpose(q, (0, 2, 1))
    qt = qt.reshape(c_dim, 8, _NB_SP, 32)
    cat = jnp.concatenate([qt[:, :, k, :] for k in range(_NB_SP)],
                          axis=-1)
    lane = jax.lax.broadcasted_iota(jnp.int32, (c_dim, 8, 128), 2)
    perm = (lane % _NB_SP) * 32 + lane // _NB_SP
    d = jnp.take_along_axis(cat, perm, axis=-1)

    ssq = jnp.sum(d * d, axis=-1, keepdims=True)
    d = d / jnp.maximum(jnp.sqrt(ssq), 1e-12)
    d = jnp.clip(d, 0.0, _CLIP)
    ssq = jnp.sum(d * d, axis=-1, keepdims=True)
    d = d / jnp.maximum(jnp.sqrt(ssq), 1e-12)
    o_ref[...] = d.reshape(1, c_dim, 1, 8, 128)


def kernel(image_batch):
    b, c, h, w = image_batch.shape
    n_h, n_w = h // _PS, w // _PS
    m = _NB_SP * n_w

    gk_big, wc_big, wr, _ = _np_consts(n_w)
    gk_big = jnp.asarray(gk_big)
    wc_big = jnp.asarray(wc_big).astype(jnp.bfloat16)
    wr = jnp.asarray(wr)

    image5 = image_batch.reshape(b, c, n_h, _PS, w)
    out = pl.pallas_call(
        _sift_body,
        grid=(b, n_h),
        in_specs=[
            pl.BlockSpec((1, c, 1, _PS, w), lambda i, j: (i, 0, j, 0, 0)),
            pl.BlockSpec((_PS, w), lambda i, j: (0, 0)),
            pl.BlockSpec((w, m), lambda i, j: (0, 0)),
            pl.BlockSpec((_NB_SP, _PS), lambda i, j: (0, 0)),
        ],
        out_specs=pl.BlockSpec((1, c, 1, n_w, 128),
                               lambda i, j: (i, 0, j, 0, 0)),
        out_shape=jax.ShapeDtypeStruct((b, c, n_h, n_w, 128), jnp.float32),
        compiler_params=pltpu.CompilerParams(
            dimension_semantics=("parallel", "arbitrary"),
        ),
        name="sift_descriptor",
    )(image5, gk_big, wc_big, wr)

    return out.reshape(b * n_h * n_w, c, _NB_ANG * _NB_SP * _NB_SP)


# speedup vs baseline: 11.9809x; 1.2768x over previous
import math

import jax
import jax.numpy as jnp
import numpy as np
from jax.experimental import pallas as pl
from jax.experimental.pallas import tpu as pltpu

_PS = 65
_NB_ANG = 8
_NB_SP = 4
_CLIP = 0.2
_EPS = 1e-10
_KS = 26
_STRIDE = 16
_PAD = 6


def _np_consts(n_w):
    x = np.arange(_PS, dtype=np.float64) - _PS // 2
    sigma = _PS / math.sqrt(2.0)
    g = np.exp(-(x * x) / (2.0 * sigma * sigma))
    g = g / g.sum()
    gk = np.outer(g, g)
    gk_big = np.tile(gk, (1, n_w)).astype(np.float32)

    ks2 = _KS / 2.0
    xc2 = ks2 - np.abs(np.arange(_KS, dtype=np.float64) + 0.5 - ks2)
    w1 = xc2 / ks2
    wmat = np.zeros((_NB_SP, _PS))
    for s in range(_NB_SP):
        start = s * _STRIDE - _PAD
        for k in range(_KS):
            j = start + k
            if 0 <= j < _PS:
                wmat[s, j] = w1[k]
    wc_big = np.zeros((n_w * _PS, n_w * _NB_SP))
    for p in range(n_w):
        wc_big[p * _PS:(p + 1) * _PS, p * _NB_SP:(p + 1) * _NB_SP] = wmat.T
    wr = wmat.astype(np.float32)

    gsum = np.kron(np.eye(n_w), np.ones((_NB_SP, _NB_SP))).astype(np.float32)
    return gk_big, wc_big.astype(np.float32), wr, gsum


_ROWS_PER_STEP = 2


def _sift_body(x_ref, gk_ref, wc_ref, wr_ref, o_ref):
    for rr in range(_ROWS_PER_STEP):
        d = _one_patch_row(x_ref[0, :, rr], gk_ref[...], wc_ref[...],
                           wr_ref[...])
        o_ref[0, :, rr] = d


def _one_patch_row(x, gk, wc, wr):
    c_dim = x.shape[0]
    w = x.shape[2]

    col = jax.lax.broadcasted_iota(jnp.int32, (_PS, w), 1)
    jj = jax.lax.rem(col, _PS)
    xr = jnp.concatenate([x[:, :, 1:], x[:, :, -1:]], axis=2)
    xl = jnp.concatenate([x[:, :, :1], x[:, :, :-1]], axis=2)
    right_val = jnp.where((jj == _PS - 1)[None], x, xr)
    left_val = jnp.where((jj == 0)[None], x, xl)
    gx = (right_val - left_val) * 0.5
    xd = jnp.concatenate([x[:, 1:, :], x[:, -1:, :]], axis=1)
    xu = jnp.concatenate([x[:, :1, :], x[:, :-1, :]], axis=1)
    gy = (xd - xu) * 0.5

    mag = jnp.sqrt(gx * gx + gy * gy + _EPS) * gk[None]

    gxe = gx + _EPS
    ax = jnp.abs(gxe)
    ay = jnp.abs(gy)
    mn = jnp.minimum(ax, ay)
    mx = jnp.maximum(ax, ay)
    r = mn / jnp.maximum(mx, 1e-30)
    r2 = r * r
    t = r * (1.2732134400
             + r2 * (-0.4235559358
                     + r2 * (0.2466700370
                             + r2 * (-0.1487908832
                                     + r2 * (0.0675856331
                                             + r2 * -0.0151248217)))))
    a1 = jnp.where(ay > ax, 2.0 - t, t)
    a2 = jnp.where(gxe < 0.0, 4.0 - a1, a1)
    u8 = jnp.where(gy < 0.0, 8.0 - a2, a2)
    bo0f = jnp.floor(u8)
    wo1 = u8 - bo0f
    b0 = jnp.where(bo0f >= _NB_ANG, bo0f - _NB_ANG, bo0f)
    c1 = wo1 * mag
    c0 = mag - c1

    pooled = {}
    for a in range(_NB_ANG):
        am1 = (a - 1) % _NB_ANG
        contrib = (jnp.where(b0 == float(a), c0, 0.0)
                   + jnp.where(b0 == float(am1), c1, 0.0))
        contrib = contrib.astype(jnp.bfloat16)
        for c in range(c_dim):
            rp = jnp.dot(wr, contrib[c], preferred_element_type=jnp.float32)
            rc = jnp.dot(rp.astype(jnp.bfloat16), wc,
                         preferred_element_type=jnp.float32)
            pooled[(a, c)] = rc
    q = jnp.stack([jnp.concatenate([pooled[(a, c)] for a in range(_NB_ANG)],
                                   axis=0) for c in range(c_dim)])

    qt = jnp.transpose(q, (0, 2, 1))
    qt = qt.reshape(c_dim, 8, _NB_SP, 32)
    cat = jnp.concatenate([qt[:, :, k, :] for k in range(_NB_SP)],
                          axis=-1)
    lane = jax.lax.broadcasted_iota(jnp.int32, (c_dim, 8, 128), 2)
    perm = (lane % _NB_SP) * 32 + lane // _NB_SP
    d = jnp.take_along_axis(cat, perm, axis=-1)

    ssq = jnp.sum(d * d, axis=-1, keepdims=True)
    d = d / jnp.maximum(jnp.sqrt(ssq), 1e-12)
    d = jnp.clip(d, 0.0, _CLIP)
    ssq = jnp.sum(d * d, axis=-1, keepdims=True)
    return d / jnp.maximum(jnp.sqrt(ssq), 1e-12)


def kernel(image_batch):
    b, c, h, w = image_batch.shape
    n_h, n_w = h // _PS, w // _PS
    m = _NB_SP * n_w

    gk_big, wc_big, wr, _ = _np_consts(n_w)
    gk_big = jnp.asarray(gk_big)
    wc_big = jnp.asarray(wc_big).astype(jnp.bfloat16)
    wr = jnp.asarray(wr).astype(jnp.bfloat16)

    image5 = image_batch.reshape(b, c, n_h, _PS, w)
    out = pl.pallas_call(
        _sift_body,
        grid=(b, n_h // _ROWS_PER_STEP),
        in_specs=[
            pl.BlockSpec((1, c, _ROWS_PER_STEP, _PS, w),
                         lambda i, j: (i, 0, j, 0, 0)),
            pl.BlockSpec((_PS, w), lambda i, j: (0, 0)),
            pl.BlockSpec((w, m), lambda i, j: (0, 0)),
            pl.BlockSpec((_NB_SP, _PS), lambda i, j: (0, 0)),
        ],
        out_specs=pl.BlockSpec((1, c, _ROWS_PER_STEP, n_w, 128),
                               lambda i, j: (i, 0, j, 0, 0)),
        out_shape=jax.ShapeDtypeStruct((b, c, n_h, n_w, 128), jnp.float32),
        compiler_params=pltpu.CompilerParams(
            dimension_semantics=("parallel", "arbitrary"),
        ),
        name="sift_descriptor",
    )(image5, gk_big, wc_big, wr)

    return out.reshape(b * n_h * n_w, c, _NB_ANG * _NB_SP * _NB_SP)


# speedup vs baseline: 12.3978x; 1.0348x over previous
import math

import jax
import jax.numpy as jnp
import numpy as np
from jax.experimental import pallas as pl
from jax.experimental.pallas import tpu as pltpu

_PS = 65
_NB_ANG = 8
_NB_SP = 4
_CLIP = 0.2
_EPS = 1e-10
_KS = 26
_STRIDE = 16
_PAD = 6


def _np_consts(n_w):
    x = np.arange(_PS, dtype=np.float64) - _PS // 2
    sigma = _PS / math.sqrt(2.0)
    g = np.exp(-(x * x) / (2.0 * sigma * sigma))
    g = g / g.sum()
    gk = np.outer(g, g)
    gk_big = np.tile(gk, (1, n_w)).astype(np.float32)

    ks2 = _KS / 2.0
    xc2 = ks2 - np.abs(np.arange(_KS, dtype=np.float64) + 0.5 - ks2)
    w1 = xc2 / ks2
    wmat = np.zeros((_NB_SP, _PS))
    for s in range(_NB_SP):
        start = s * _STRIDE - _PAD
        for k in range(_KS):
            j = start + k
            if 0 <= j < _PS:
                wmat[s, j] = w1[k]
    wc_big = np.zeros((n_w * _PS, n_w * _NB_SP))
    for p in range(n_w):
        wc_big[p * _PS:(p + 1) * _PS, p * _NB_SP:(p + 1) * _NB_SP] = wmat.T
    wr = wmat.astype(np.float32)

    gsum = np.kron(np.eye(n_w), np.ones((_NB_SP, _NB_SP))).astype(np.float32)
    return gk_big, wc_big.astype(np.float32), wr, gsum


_ROWS_PER_STEP = 2


def _sift_body(x_ref, gk_ref, wc_ref, wr_ref, o_ref):
    c_dim = o_ref.shape[1]
    for rr in range(_ROWS_PER_STEP):
        q = jnp.stack([_one_unit(x_ref[0, c, rr], gk_ref[...], wc_ref[...],
                                 wr_ref[...]) for c in range(c_dim)])
        qt = jnp.transpose(q, (0, 2, 1))
        qt = qt.reshape(c_dim, 8, _NB_SP, 32)
        cat = jnp.concatenate([qt[:, :, k, :] for k in range(_NB_SP)],
                              axis=-1)
        lane = jax.lax.broadcasted_iota(jnp.int32, (c_dim, 8, 128), 2)
        perm = (lane % _NB_SP) * 32 + lane // _NB_SP
        d = jnp.take_along_axis(cat, perm, axis=-1)

        ssq = jnp.sum(d * d, axis=-1, keepdims=True)
        d = d / jnp.maximum(jnp.sqrt(ssq), 1e-12)
        d = jnp.clip(d, 0.0, _CLIP)
        ssq = jnp.sum(d * d, axis=-1, keepdims=True)
        d = d / jnp.maximum(jnp.sqrt(ssq), 1e-12)
        o_ref[0, :, rr] = d


def _one_unit(x, gk, wc, wr):
    w = x.shape[1]

    col = jax.lax.broadcasted_iota(jnp.int32, (_PS, w), 1)
    jj = jax.lax.rem(col, _PS)
    xr = jnp.concatenate([x[:, 1:], x[:, -1:]], axis=1)
    xl = jnp.concatenate([x[:, :1], x[:, :-1]], axis=1)
    right_val = jnp.where(jj == _PS - 1, x, xr)
    left_val = jnp.where(jj == 0, x, xl)
    gx = (right_val - left_val) * 0.5
    xd = jnp.concatenate([x[1:, :], x[-1:, :]], axis=0)
    xu = jnp.concatenate([x[:1, :], x[:-1, :]], axis=0)
    gy = (xd - xu) * 0.5

    mag = jnp.sqrt(gx * gx + gy * gy + _EPS) * gk

    gxe = gx + _EPS
    ax = jnp.abs(gxe)
    ay = jnp.abs(gy)
    mn = jnp.minimum(ax, ay)
    mx = jnp.maximum(ax, ay)
    r = mn / jnp.maximum(mx, 1e-30)
    r2 = r * r
    t = r * (1.2732134400
             + r2 * (-0.4235559358
                     + r2 * (0.2466700370
                             + r2 * (-0.1487908832
                                     + r2 * (0.0675856331
                                             + r2 * -0.0151248217)))))
    a1 = jnp.where(ay > ax, 2.0 - t, t)
    a2 = jnp.where(gxe < 0.0, 4.0 - a1, a1)
    u8 = jnp.where(gy < 0.0, 8.0 - a2, a2)
    bo0f = jnp.floor(u8)
    wo1 = u8 - bo0f
    b0 = jnp.where(bo0f >= _NB_ANG, bo0f - _NB_ANG, bo0f)
    c1 = wo1 * mag
    c0 = mag - c1

    b0h = b0.astype(jnp.bfloat16)
    c0h = c0.astype(jnp.bfloat16)
    c1h = c1.astype(jnp.bfloat16)
    zero = jnp.zeros_like(c0h)
    eq = [b0h == jnp.bfloat16(float(a)) for a in range(_NB_ANG)]
    rcs = []
    for a in range(_NB_ANG):
        am1 = (a - 1) % _NB_ANG
        contrib = jnp.where(eq[a], c0h, jnp.where(eq[am1], c1h, zero))
        rp = jnp.dot(wr, contrib, preferred_element_type=jnp.float32)
        rcs.append(jnp.dot(rp.astype(jnp.bfloat16), wc,
                           preferred_element_type=jnp.float32))
    return jnp.concatenate(rcs, axis=0)


def kernel(image_batch):
    b, c, h, w = image_batch.shape
    n_h, n_w = h // _PS, w // _PS
    m = _NB_SP * n_w

    gk_big, wc_big, wr, _ = _np_consts(n_w)
    gk_big = jnp.asarray(gk_big)
    wc_big = jnp.asarray(wc_big).astype(jnp.bfloat16)
    wr = jnp.asarray(wr).astype(jnp.bfloat16)

    image5 = image_batch.reshape(b, c, n_h, _PS, w)
    out = pl.pallas_call(
        _sift_body,
        grid=(b, n_h // _ROWS_PER_STEP),
        in_specs=[
            pl.BlockSpec((1, c, _ROWS_PER_STEP, _PS, w),
                         lambda i, j: (i, 0, j, 0, 0)),
            pl.BlockSpec((_PS, w), lambda i, j: (0, 0)),
            pl.BlockSpec((w, m), lambda i, j: (0, 0)),
            pl.BlockSpec((_NB_SP, _PS), lambda i, j: (0, 0)),
        ],
        out_specs=pl.BlockSpec((1, c, _ROWS_PER_STEP, n_w, 128),
                               lambda i, j: (i, 0, j, 0, 0)),
        out_shape=jax.ShapeDtypeStruct((b, c, n_h, n_w, 128), jnp.float32),
        compiler_params=pltpu.CompilerParams(
            dimension_semantics=("parallel", "arbitrary"),
        ),
        name="sift_descriptor",
    )(image5, gk_big, wc_big, wr)

    return out.reshape(b * n_h * n_w, c, _NB_ANG * _NB_SP * _NB_SP)


# speedup vs baseline: 13.7387x; 1.1082x over previous
import math

import jax
import jax.numpy as jnp
import numpy as np
from jax.experimental import pallas as pl
from jax.experimental.pallas import tpu as pltpu

_PS = 65
_NB_ANG = 8
_NB_SP = 4
_CLIP = 0.2
_EPS = 1e-10
_KS = 26
_STRIDE = 16
_PAD = 6


def _np_consts(n_w):
    x = np.arange(_PS, dtype=np.float64) - _PS // 2
    sigma = _PS / math.sqrt(2.0)
    g = np.exp(-(x * x) / (2.0 * sigma * sigma))
    g = g / g.sum()
    gk = np.outer(g, g)
    gk_big = np.tile(gk, (1, n_w)).astype(np.float32)

    ks2 = _KS / 2.0
    xc2 = ks2 - np.abs(np.arange(_KS, dtype=np.float64) + 0.5 - ks2)
    w1 = xc2 / ks2
    wmat = np.zeros((_NB_SP, _PS))
    for s in range(_NB_SP):
        start = s * _STRIDE - _PAD
        for k in range(_KS):
            j = start + k
            if 0 <= j < _PS:
                wmat[s, j] = w1[k]
    wc_big = np.zeros((n_w * _PS, n_w * _NB_SP))
    for p in range(n_w):
        wc_big[p * _PS:(p + 1) * _PS, p * _NB_SP:(p + 1) * _NB_SP] = wmat.T
    wr = wmat.astype(np.float32)

    gsum = np.kron(np.eye(n_w), np.ones((_NB_SP, _NB_SP))).astype(np.float32)
    return gk_big, wc_big.astype(np.float32), wr, gsum


def _sift_body(x_ref, gk_ref, wc_ref, wr_ref, o_ref):
    n_h = o_ref.shape[2]
    q = jnp.stack([_one_unit(x_ref[0, 0, ph * _PS:(ph + 1) * _PS, :],
                             gk_ref[...], wc_ref[...], wr_ref[...])
                   for ph in range(n_h)])
    qt = jnp.transpose(q, (0, 2, 1))
    qt = qt.reshape(n_h, 8, _NB_SP, 32)
    cat = jnp.concatenate([qt[:, :, k, :] for k in range(_NB_SP)],
                          axis=-1)
    lane = jax.lax.broadcasted_iota(jnp.int32, (n_h, 8, 128), 2)
    perm = (lane % _NB_SP) * 32 + lane // _NB_SP
    d = jnp.take_along_axis(cat, perm, axis=-1)

    ssq = jnp.sum(d * d, axis=-1, keepdims=True)
    d = d / jnp.maximum(jnp.sqrt(ssq), 1e-12)
    d = jnp.clip(d, 0.0, _CLIP)
    ssq = jnp.sum(d * d, axis=-1, keepdims=True)
    d = d / jnp.maximum(jnp.sqrt(ssq), 1e-12)
    o_ref[0, 0] = d


def _one_unit(x, gk, wc, wr):
    w = x.shape[1]

    col = jax.lax.broadcasted_iota(jnp.int32, (_PS, w), 1)
    jj = jax.lax.rem(col, _PS)
    xr = jnp.concatenate([x[:, 1:], x[:, -1:]], axis=1)
    xl = jnp.concatenate([x[:, :1], x[:, :-1]], axis=1)
    right_val = jnp.where(jj == _PS - 1, x, xr)
    left_val = jnp.where(jj == 0, x, xl)
    gx = (right_val - left_val) * 0.5
    xd = jnp.concatenate([x[1:, :], x[-1:, :]], axis=0)
    xu = jnp.concatenate([x[:1, :], x[:-1, :]], axis=0)
    gy = (xd - xu) * 0.5

    mag = jnp.sqrt(gx * gx + gy * gy + _EPS) * gk

    gxe = gx + _EPS
    ax = jnp.abs(gxe)
    ay = jnp.abs(gy)
    mn = jnp.minimum(ax, ay)
    mx = jnp.maximum(ax, ay)
    r = mn / jnp.maximum(mx, 1e-30)
    r2 = r * r
    t = r * (1.2732134400
             + r2 * (-0.4235559358
                     + r2 * (0.2466700370
                             + r2 * (-0.1487908832
                                     + r2 * (0.0675856331
                                             + r2 * -0.0151248217)))))
    a1 = jnp.where(ay > ax, 2.0 - t, t)
    a2 = jnp.where(gxe < 0.0, 4.0 - a1, a1)
    u8 = jnp.where(gy < 0.0, 8.0 - a2, a2)
    bo0f = jnp.floor(u8)
    wo1 = u8 - bo0f
    b0 = jnp.where(bo0f >= _NB_ANG, bo0f - _NB_ANG, bo0f)
    c1 = wo1 * mag
    c0 = mag - c1

    b0h = b0.astype(jnp.bfloat16)
    c0h = c0.astype(jnp.bfloat16)
    c1h = c1.astype(jnp.bfloat16)
    zero = jnp.zeros_like(c0h)
    eq = [b0h == jnp.bfloat16(float(a)) for a in range(_NB_ANG)]
    rcs = []
    for a in range(_NB_ANG):
        am1 = (a - 1) % _NB_ANG
        contrib = jnp.where(eq[a], c0h, jnp.where(eq[am1], c1h, zero))
        rp = jnp.dot(wr, contrib, preferred_element_type=jnp.float32)
        rcs.append(jnp.dot(rp.astype(jnp.bfloat16), wc,
                           preferred_element_type=jnp.float32))
    return jnp.concatenate(rcs, axis=0)


def kernel(image_batch):
    b, c, h, w = image_batch.shape
    n_h, n_w = h // _PS, w // _PS
    m = _NB_SP * n_w

    gk_big, wc_big, wr, _ = _np_consts(n_w)
    gk_big = jnp.asarray(gk_big)
    wc_big = jnp.asarray(wc_big).astype(jnp.bfloat16)
    wr = jnp.asarray(wr).astype(jnp.bfloat16)

    out = pl.pallas_call(
        _sift_body,
        grid=(b, c),
        in_specs=[
            pl.BlockSpec((1, 1, h, w), lambda i, ci: (i, ci, 0, 0)),
            pl.BlockSpec((_PS, w), lambda i, ci: (0, 0)),
            pl.BlockSpec((w, m), lambda i, ci: (0, 0)),
            pl.BlockSpec((_NB_SP, _PS), lambda i, ci: (0, 0)),
        ],
        out_specs=pl.BlockSpec((1, 1, n_h, n_w, 128),
                               lambda i, ci: (i, ci, 0, 0, 0)),
        out_shape=jax.ShapeDtypeStruct((b, c, n_h, n_w, 128), jnp.float32),
        compiler_params=pltpu.CompilerParams(
            dimension_semantics=("parallel", "parallel"),
        ),
        name="sift_descriptor",
    )(image_batch, gk_big, wc_big, wr)

    return out.reshape(b * n_h * n_w, c, _NB_ANG * _NB_SP * _NB_SP)


# speedup vs baseline: 29.2501x; 2.1290x over previous
import math

import jax
import jax.numpy as jnp
import numpy as np
from jax.experimental import pallas as pl
from jax.experimental.pallas import tpu as pltpu

_PS = 65
_NB_ANG = 8
_NB_SP = 4
_CLIP = 0.2
_EPS = 1e-10
_KS = 26
_STRIDE = 16
_PAD = 6


def _np_consts(n_h, n_w):
    x = np.arange(_PS, dtype=np.float64) - _PS // 2
    sigma = _PS / math.sqrt(2.0)
    g = np.exp(-(x * x) / (2.0 * sigma * sigma))
    g = g / g.sum()
    gk = np.outer(g, g)
    gk_big = np.tile(gk, (n_h, n_w)).astype(np.float32)

    ks2 = _KS / 2.0
    xc2 = ks2 - np.abs(np.arange(_KS, dtype=np.float64) + 0.5 - ks2)
    w1 = xc2 / ks2
    wmat = np.zeros((_NB_SP, _PS))
    for s in range(_NB_SP):
        start = s * _STRIDE - _PAD
        for k in range(_KS):
            j = start + k
            if 0 <= j < _PS:
                wmat[s, j] = w1[k]
    wc_big = np.zeros((n_w * _PS, n_w * _NB_SP))
    for p in range(n_w):
        wc_big[p * _PS:(p + 1) * _PS, p * _NB_SP:(p + 1) * _NB_SP] = wmat.T
    wr_big = np.zeros((n_h * _NB_SP, n_h * _PS))
    for p in range(n_h):
        wr_big[p * _NB_SP:(p + 1) * _NB_SP, p * _PS:(p + 1) * _PS] = wmat
    return gk_big, wc_big.astype(np.float32), wr_big.astype(np.float32)


def _sift_body(x_ref, gk_ref, wc_ref, wr_ref, o_ref):
    n_h, n_w = o_ref.shape[2], o_ref.shape[3]
    x = x_ref[0, 0]
    hh, w = x.shape

    col = jax.lax.broadcasted_iota(jnp.int32, (hh, w), 1)
    jj = jax.lax.rem(col, _PS)
    row = jax.lax.broadcasted_iota(jnp.int32, (hh, w), 0)
    ii = jax.lax.rem(row, _PS)
    xr = jnp.concatenate([x[:, 1:], x[:, -1:]], axis=1)
    xl = jnp.concatenate([x[:, :1], x[:, :-1]], axis=1)
    right_val = jnp.where(jj == _PS - 1, x, xr)
    left_val = jnp.where(jj == 0, x, xl)
    gx = (right_val - left_val) * 0.5
    xd = jnp.concatenate([x[1:, :], x[-1:, :]], axis=0)
    xu = jnp.concatenate([x[:1, :], x[:-1, :]], axis=0)
    down_val = jnp.where(ii == _PS - 1, x, xd)
    up_val = jnp.where(ii == 0, x, xu)
    gy = (down_val - up_val) * 0.5

    mag = jnp.sqrt(gx * gx + gy * gy + _EPS) * gk_ref[...]

    gxe = gx + _EPS
    ax = jnp.abs(gxe)
    ay = jnp.abs(gy)
    mn = jnp.minimum(ax, ay)
    mx = jnp.maximum(ax, ay)
    r = mn / jnp.maximum(mx, 1e-30)
    r2 = r * r
    t = r * (1.2732134400
             + r2 * (-0.4235559358
                     + r2 * (0.2466700370
                             + r2 * (-0.1487908832
                                     + r2 * (0.0675856331
                                             + r2 * -0.0151248217)))))
    a1 = jnp.where(ay > ax, 2.0 - t, t)
    a2 = jnp.where(gxe < 0.0, 4.0 - a1, a1)
    u8 = jnp.where(gy < 0.0, 8.0 - a2, a2)
    bo0f = jnp.floor(u8)
    wo1 = u8 - bo0f
    b0 = jnp.where(bo0f >= _NB_ANG, bo0f - _NB_ANG, bo0f)
    c1 = wo1 * mag
    c0 = mag - c1

    b0h = b0.astype(jnp.bfloat16)
    c0h = c0.astype(jnp.bfloat16)
    c1h = c1.astype(jnp.bfloat16)
    zero = jnp.zeros_like(c0h)
    eq = [b0h == jnp.bfloat16(float(a)) for a in range(_NB_ANG)]
    wc = wc_ref[...]
    wrb = wr_ref[...]
    qas = []
    for a in range(_NB_ANG):
        am1 = (a - 1) % _NB_ANG
        contrib = jnp.where(eq[a], c0h, jnp.where(eq[am1], c1h, zero))
        rp = jnp.dot(wrb, contrib, preferred_element_type=jnp.float32)
        qas.append(jnp.dot(rp.astype(jnp.bfloat16), wc,
                           preferred_element_type=jnp.float32))
    qa = jnp.stack(qas)

    qa = qa.reshape(_NB_ANG, n_h, _NB_SP, _NB_SP * n_w)
    q = jnp.transpose(qa, (1, 0, 2, 3)).reshape(n_h, _NB_ANG * _NB_SP,
                                                _NB_SP * n_w)
    qt = jnp.transpose(q, (0, 2, 1))
    qt = qt.reshape(n_h, n_w, _NB_SP, 32)
    cat = jnp.concatenate([qt[:, :, k, :] for k in range(_NB_SP)],
                          axis=-1)
    lane = jax.lax.broadcasted_iota(jnp.int32, (n_h, n_w, 128), 2)
    perm = (lane % _NB_SP) * 32 + lane // _NB_SP
    d = jnp.take_along_axis(cat, perm, axis=-1)

    ssq = jnp.sum(d * d, axis=-1, keepdims=True)
    d = d / jnp.maximum(jnp.sqrt(ssq), 1e-12)
    d = jnp.clip(d, 0.0, _CLIP)
    ssq = jnp.sum(d * d, axis=-1, keepdims=True)
    d = d / jnp.maximum(jnp.sqrt(ssq), 1e-12)
    o_ref[0, 0] = d


def kernel(image_batch):
    b, c, h, w = image_batch.shape
    n_h, n_w = h // _PS, w // _PS
    m = _NB_SP * n_w

    gk_big, wc_big, wr_big = _np_consts(n_h, n_w)
    gk_big = jnp.asarray(gk_big)
    wc_big = jnp.asarray(wc_big).astype(jnp.bfloat16)
    wr_big = jnp.asarray(wr_big).astype(jnp.bfloat16)

    out = pl.pallas_call(
        _sift_body,
        grid=(b, c),
        in_specs=[
            pl.BlockSpec((1, 1, h, w), lambda i, ci: (i, ci, 0, 0)),
            pl.BlockSpec((h, w), lambda i, ci: (0, 0)),
            pl.BlockSpec((w, m), lambda i, ci: (0, 0)),
            pl.BlockSpec((_NB_SP * n_h, h), lambda i, ci: (0, 0)),
        ],
        out_specs=pl.BlockSpec((1, 1, n_h, n_w, 128),
                               lambda i, ci: (i, ci, 0, 0, 0)),
        out_shape=jax.ShapeDtypeStruct((b, c, n_h, n_w, 128), jnp.float32),
        compiler_params=pltpu.CompilerParams(
            dimension_semantics=("parallel", "parallel"),
        ),
        name="sift_descriptor",
    )(image_batch, gk_big, wc_big, wr_big)

    return out.reshape(b * n_h * n_w, c, _NB_ANG * _NB_SP * _NB_SP)


# speedup vs baseline: 30.6035x; 1.0463x over previous
import math

import jax
import jax.numpy as jnp
import numpy as np
from jax.experimental import pallas as pl
from jax.experimental.pallas import tpu as pltpu

_PS = 65
_NB_ANG = 8
_NB_SP = 4
_CLIP = 0.2
_EPS = 1e-10
_KS = 26
_STRIDE = 16
_PAD = 6


def _np_consts(n_h, n_w):
    x = np.arange(_PS, dtype=np.float64) - _PS // 2
    sigma = _PS / math.sqrt(2.0)
    g = np.exp(-(x * x) / (2.0 * sigma * sigma))
    g = g / g.sum()

    ks2 = _KS / 2.0
    xc2 = ks2 - np.abs(np.arange(_KS, dtype=np.float64) + 0.5 - ks2)
    w1 = xc2 / ks2
    wmat = np.zeros((_NB_SP, _PS))
    for s in range(_NB_SP):
        start = s * _STRIDE - _PAD
        for k in range(_KS):
            j = start + k
            if 0 <= j < _PS:
                wmat[s, j] = w1[k]
    wcm = wmat.T * (0.5 * g[:, None])
    wc_big = np.zeros((n_w * _PS, n_w * _NB_SP))
    for p in range(n_w):
        wc_big[p * _PS:(p + 1) * _PS, p * _NB_SP:(p + 1) * _NB_SP] = wcm
    wrm = wmat * g[None, :]
    wr_big = np.zeros((n_h * _NB_SP, n_h * _PS))
    for p in range(n_h):
        wr_big[p * _NB_SP:(p + 1) * _NB_SP, p * _PS:(p + 1) * _PS] = wrm
    return wc_big.astype(np.float32), wr_big.astype(np.float32)


def _sift_body(x_ref, wc_ref, wr_ref, o_ref):
    n_h, n_w = o_ref.shape[2], o_ref.shape[3]
    x = x_ref[0, 0]
    hh, w = x.shape

    col = jax.lax.broadcasted_iota(jnp.int32, (hh, w), 1)
    jj = jax.lax.rem(col, _PS)
    row = jax.lax.broadcasted_iota(jnp.int32, (hh, w), 0)
    ii = jax.lax.rem(row, _PS)
    xr = jnp.concatenate([x[:, 1:], x[:, -1:]], axis=1)
    xl = jnp.concatenate([x[:, :1], x[:, :-1]], axis=1)
    right_val = jnp.where(jj == _PS - 1, x, xr)
    left_val = jnp.where(jj == 0, x, xl)
    gx = right_val - left_val
    xd = jnp.concatenate([x[1:, :], x[-1:, :]], axis=0)
    xu = jnp.concatenate([x[:1, :], x[:-1, :]], axis=0)
    down_val = jnp.where(ii == _PS - 1, x, xd)
    up_val = jnp.where(ii == 0, x, xu)
    gy = down_val - up_val

    mag = jnp.sqrt(gx * gx + gy * gy + 4.0 * _EPS)

    gxe = gx + 2.0 * _EPS
    ax = jnp.abs(gxe)
    ay = jnp.abs(gy)
    mn = jnp.minimum(ax, ay)
    mx = jnp.maximum(ax, ay)
    r = mn / jnp.maximum(mx, 1e-30)
    r2 = r * r
    t = r * (1.2730840300
             + r2 * (-0.4207247425
                     + r2 * (0.2299685627
                             + r2 * (-0.1092053987
                                     + r2 * 0.0268949366))))
    a1 = jnp.where(ay > ax, 2.0 - t, t)
    a2 = jnp.where(gxe < 0.0, 4.0 - a1, a1)
    u8 = jnp.where(gy < 0.0, 8.0 - a2, a2)
    bo0f = jnp.floor(u8)
    wo1 = u8 - bo0f
    b0 = jnp.where(bo0f >= _NB_ANG, bo0f - _NB_ANG, bo0f)
    c1 = wo1 * mag
    c0 = mag - c1

    b0h = b0.astype(jnp.bfloat16)
    c0h = c0.astype(jnp.bfloat16)
    c1h = c1.astype(jnp.bfloat16)
    zero = jnp.zeros_like(c0h)
    eq = [b0h == jnp.bfloat16(float(a)) for a in range(_NB_ANG)]
    wc = wc_ref[...]
    wrb = wr_ref[...]
    qas = []
    for a in range(_NB_ANG):
        am1 = (a - 1) % _NB_ANG
        contrib = jnp.where(eq[a], c0h, jnp.where(eq[am1], c1h, zero))
        rp = jnp.dot(wrb, contrib, preferred_element_type=jnp.float32)
        qas.append(jnp.dot(rp.astype(jnp.bfloat16), wc,
                           preferred_element_type=jnp.float32))
    qa = jnp.stack(qas)

    qa = qa.reshape(_NB_ANG, n_h, _NB_SP, _NB_SP * n_w)
    q = jnp.transpose(qa, (1, 0, 2, 3)).reshape(n_h, _NB_ANG * _NB_SP,
                                                _NB_SP * n_w)
    qt = jnp.transpose(q, (0, 2, 1))
    qt = qt.reshape(n_h, n_w, _NB_SP, 32)
    cat = jnp.concatenate([qt[:, :, k, :] for k in range(_NB_SP)],
                          axis=-1)
    lane = jax.lax.broadcasted_iota(jnp.int32, (n_h, n_w, 128), 2)
    perm = (lane % _NB_SP) * 32 + lane // _NB_SP
    d = jnp.take_along_axis(cat, perm, axis=-1)

    ssq = jnp.sum(d * d, axis=-1, keepdims=True)
    d = d / jnp.maximum(jnp.sqrt(ssq), 1e-12)
    d = jnp.clip(d, 0.0, _CLIP)
    ssq = jnp.sum(d * d, axis=-1, keepdims=True)
    d = d / jnp.maximum(jnp.sqrt(ssq), 1e-12)
    o_ref[0, 0] = d


def kernel(image_batch):
    b, c, h, w = image_batch.shape
    n_h, n_w = h // _PS, w // _PS
    m = _NB_SP * n_w

    wc_big, wr_big = _np_consts(n_h, n_w)
    wc_big = jnp.asarray(wc_big).astype(jnp.bfloat16)
    wr_big = jnp.asarray(wr_big).astype(jnp.bfloat16)

    out = pl.pallas_call(
        _sift_body,
        grid=(b, c),
        in_specs=[
            pl.BlockSpec((1, 1, h, w), lambda i, ci: (i, ci, 0, 0)),
            pl.BlockSpec((w, m), lambda i, ci: (0, 0)),
            pl.BlockSpec((_NB_SP * n_h, h), lambda i, ci: (0, 0)),
        ],
        out_specs=pl.BlockSpec((1, 1, n_h, n_w, 128),
                               lambda i, ci: (i, ci, 0, 0, 0)),
        out_shape=jax.ShapeDtypeStruct((b, c, n_h, n_w, 128), jnp.float32),
        compiler_params=pltpu.CompilerParams(
            dimension_semantics=("parallel", "parallel"),
        ),
        name="sift_descriptor",
    )(image_batch, wc_big, wr_big)

    return out.reshape(b * n_h * n_w, c, _NB_ANG * _NB_SP * _NB_SP)
